# baseline (speedup 1.0000x reference)
import functools

import jax
import jax.numpy as jnp
from jax import lax
from jax.experimental import pallas as pl
from jax.experimental.pallas import tpu as pltpu
from jax.experimental.pallas import tpu_sc as plsc

B = 16384
D = 32
TOTAL = 131072
NSAMP = 6
NC, NS = 2, 16
NW = NC * NS
CHUNK = 128
POS_PER_W = TOTAL // NW
NCHUNK = POS_PER_W // CHUNK
VTOT = B * NSAMP
V_PER_W = VTOT // NW
NVCHUNK = V_PER_W // CHUNK
SEG_PER_TILE = B // NS


def _sc_kernel(bag2d, offsets, v2d, u_weight, v_weight):
  mesh = plsc.VectorSubcoreMesh(core_axis_name="c", subcore_axis_name="s")

  @functools.partial(
      pl.kernel,
      out_type=(
          jax.ShapeDtypeStruct((NC, B, D), jnp.float32),
          jax.ShapeDtypeStruct((NC, B), jnp.float32),
          jax.ShapeDtypeStruct((VTOT, D), jnp.float32),
      ),
      mesh=mesh,
      compiler_params=pltpu.CompilerParams(needs_layout_passes=False,
                                           use_tc_tiling_on_sc=False),
      scratch_types=(
          pltpu.VMEM((B,), jnp.int32),
          pltpu.VMEM((NCHUNK, CHUNK), jnp.int32),
          pltpu.VMEM((NCHUNK, CHUNK), jnp.int32),
          pltpu.VMEM((NVCHUNK, CHUNK), jnp.int32),
          pltpu.VMEM((CHUNK, D), jnp.float32),
          pltpu.VMEM((CHUNK, D), jnp.float32),
          pltpu.VMEM((CHUNK,), jnp.float32),
          pltpu.VMEM((256, D), jnp.float32),
          pltpu.VMEM((SEG_PER_TILE,), jnp.float32),
          pltpu.VMEM_SHARED((B, D), jnp.float32),
          pltpu.VMEM_SHARED((B,), jnp.float32),
          pltpu.SemaphoreType.DMA,
      ),
  )
  def body(bag_hbm, off_hbm, v_hbm, uw_hbm, vw_hbm,
           psum_hbm, pcnt_hbm, embv_hbm,
           off_v, idx_v, seg_v, vidx_v, rows_v, vrows_v, ones_v,
           stage2d_v, stage1d_v, acc_s, cnt_s, sem):
    c = lax.axis_index("c")
    s = lax.axis_index("s")
    wid = s * NC + c

    zf = jnp.zeros((16,), jnp.float32)
    onef = jnp.full((16,), 1.0, jnp.float32)

    pltpu.sync_copy(off_hbm, off_v)
    pltpu.sync_copy(bag_hbm.at[pl.ds(wid * NCHUNK, NCHUNK)], idx_v)
    pltpu.sync_copy(v_hbm.at[pl.ds(wid * NVCHUNK, NVCHUNK)], vidx_v)

    for i in range(SEG_PER_TILE // 16):
      stage1d_v[pl.ds(i * 16, 16)] = zf

    def zrow(i, carry):
      stage2d_v[i, pl.ds(0, 16)] = zf
      stage2d_v[i, pl.ds(16, 16)] = zf
      return carry
    lax.fori_loop(0, 256, zrow, 0)

    for i in range(CHUNK // 16):
      ones_v[pl.ds(i * 16, 16)] = onef

    pltpu.sync_copy(stage1d_v, cnt_s.at[pl.ds(s * SEG_PER_TILE, SEG_PER_TILE)])
    for k in range(SEG_PER_TILE // 256):
      pltpu.sync_copy(stage2d_v,
                      acc_s.at[pl.ds(s * SEG_PER_TILE + k * 256, 256)])

    lane = lax.iota(jnp.int32, 16)

    def seg_chunk(j, carry):
      base = wid * POS_PER_W + j * CHUNK
      for k in range(CHUNK // 16):
        pos = base + k * 16 + lane
        lo = jnp.zeros((16,), jnp.int32)
        sz = B // 2
        while sz >= 1:
          cand = lo + sz
          oc = plsc.load_gather(off_v, [cand])
          lo = jnp.where(oc <= pos, cand, lo)
          sz //= 2
        seg_v[j, pl.ds(k * 16, 16)] = lo
      return carry
    lax.fori_loop(0, NCHUNK, seg_chunk, 0)

    plsc.subcore_barrier()

    def bag_chunk(j, carry):
      pltpu.async_copy(uw_hbm.at[idx_v.at[j]], rows_v, sem).wait()
      pltpu.sync_copy(rows_v, acc_s.at[seg_v.at[j]], add=True)
      pltpu.sync_copy(ones_v, cnt_s.at[seg_v.at[j]], add=True)
      return carry
    lax.fori_loop(0, NCHUNK, bag_chunk, 0)

    def v_chunk(j, carry):
      pltpu.async_copy(vw_hbm.at[vidx_v.at[j]], vrows_v, sem).wait()
      pltpu.sync_copy(vrows_v, embv_hbm.at[pl.ds(wid * V_PER_W + j * CHUNK,
                                                 CHUNK)])
      return carry
    lax.fori_loop(0, NVCHUNK, v_chunk, 0)

    plsc.subcore_barrier()

    pltpu.sync_copy(cnt_s.at[pl.ds(s * SEG_PER_TILE, SEG_PER_TILE)], stage1d_v)
    pltpu.sync_copy(stage1d_v, pcnt_hbm.at[c, pl.ds(s * SEG_PER_TILE,
                                                    SEG_PER_TILE)])
    for k in range(SEG_PER_TILE // 256):
      off0 = s * SEG_PER_TILE + k * 256
      pltpu.sync_copy(acc_s.at[pl.ds(off0, 256)], stage2d_v)
      pltpu.sync_copy(stage2d_v, psum_hbm.at[c, pl.ds(off0, 256)])

  return body(bag2d, offsets, v2d, u_weight, v_weight)


TBLK = 8192


def _tr_body(in_ref, out_ref):
  out_ref[...] = in_ref[...].T


def _tc_transpose(table_t):
  V = table_t.shape[1]
  nblk = (V + TBLK - 1) // TBLK
  return pl.pallas_call(
      _tr_body,
      grid=(nblk,),
      in_specs=[pl.BlockSpec((D, TBLK), lambda i: (0, i))],
      out_specs=pl.BlockSpec((TBLK, D), lambda i: (i, 0)),
      out_shape=jax.ShapeDtypeStruct((V, D), jnp.float32),
  )(table_t)


BLK = 1024


def _tc_body(ps_ref, pc_ref, ev_ref, out_ref):
  ps = ps_ref[...]
  pc = pc_ref[...]
  ev = ev_ref[...]
  sums = ps[0] + ps[1]
  cnt = pc[0] + pc[1]
  emb_u = sums / jnp.maximum(cnt, 1.0)[:, None]
  out_ref[...] = jnp.sum(emb_u[:, None, :] * ev, axis=-1)


def _tc_dot(psum, pcnt, embv3):
  nblk = B // BLK
  return pl.pallas_call(
      _tc_body,
      grid=(nblk,),
      in_specs=[
          pl.BlockSpec((NC, BLK, D), lambda i: (0, i, 0)),
          pl.BlockSpec((NC, BLK), lambda i: (0, i)),
          pl.BlockSpec((BLK, NSAMP, D), lambda i: (i, 0, 0)),
      ],
      out_specs=pl.BlockSpec((BLK, NSAMP), lambda i: (i, 0)),
      out_shape=jax.ShapeDtypeStruct((B, NSAMP), jnp.float32),
  )(psum, pcnt, embv3)


@jax.jit
def kernel(bag, offsets, v, u_weight, v_weight):
  bag2d = bag.astype(jnp.int32).reshape(NW * NCHUNK, CHUNK)
  v2d = v.astype(jnp.int32).reshape(NW * NVCHUNK, CHUNK)
  uw = _tc_transpose(u_weight.T)
  vw = _tc_transpose(v_weight.T)
  psum, pcnt, embv = _sc_kernel(bag2d, offsets.astype(jnp.int32), v2d,
                                uw, vw)
  return _tc_dot(psum, pcnt, embv.reshape(B, NSAMP, D))


# speedup vs baseline: 1.8899x; 1.8899x over previous
import functools

import jax
import jax.numpy as jnp
from jax import lax
from jax.experimental import pallas as pl
from jax.experimental.pallas import tpu as pltpu
from jax.experimental.pallas import tpu_sc as plsc

B = 16384
D = 32
TOTAL = 131072
NSAMP = 6
NC, NS = 2, 16
NW = NC * NS
CHUNK = 128
POS_PER_W = TOTAL // NW
NCHUNK = POS_PER_W // CHUNK
VTOT = B * NSAMP
V_PER_W = VTOT // NW
NVCHUNK = V_PER_W // CHUNK
SEG_PER_TILE = B // NS


def _sc_kernel(bag2d, offsets, v2d, u_weight, v_weight):
  mesh = plsc.VectorSubcoreMesh(core_axis_name="c", subcore_axis_name="s")

  @functools.partial(
      pl.kernel,
      out_type=(
          jax.ShapeDtypeStruct((NC, B, D), jnp.float32),
          jax.ShapeDtypeStruct((NC, B), jnp.float32),
          jax.ShapeDtypeStruct((VTOT, D), jnp.float32),
      ),
      mesh=mesh,
      compiler_params=pltpu.CompilerParams(needs_layout_passes=False,
                                           use_tc_tiling_on_sc=False),
      scratch_types=(
          pltpu.VMEM((B,), jnp.int32),
          pltpu.VMEM((NCHUNK, CHUNK), jnp.int32),
          pltpu.VMEM((NCHUNK, CHUNK), jnp.int32),
          pltpu.VMEM((NVCHUNK, CHUNK), jnp.int32),
          pltpu.VMEM((CHUNK, D), jnp.float32),
          pltpu.VMEM((CHUNK, D), jnp.float32),
          pltpu.VMEM((CHUNK,), jnp.float32),
          pltpu.VMEM((256, D), jnp.float32),
          pltpu.VMEM((SEG_PER_TILE,), jnp.float32),
          pltpu.VMEM_SHARED((B, D), jnp.float32),
          pltpu.VMEM_SHARED((B,), jnp.float32),
          pltpu.SemaphoreType.DMA,
      ),
  )
  def body(bag_hbm, off_hbm, v_hbm, uw_hbm, vw_hbm,
           psum_hbm, pcnt_hbm, embv_hbm,
           off_v, idx_v, seg_v, vidx_v, rows_v, vrows_v, ones_v,
           stage2d_v, stage1d_v, acc_s, cnt_s, sem):
    c = lax.axis_index("c")
    s = lax.axis_index("s")
    wid = s * NC + c

    zf = jnp.zeros((16,), jnp.float32)
    onef = jnp.full((16,), 1.0, jnp.float32)

    pltpu.sync_copy(off_hbm, off_v)
    pltpu.sync_copy(bag_hbm.at[pl.ds(wid * NCHUNK, NCHUNK)], idx_v)
    pltpu.sync_copy(v_hbm.at[pl.ds(wid * NVCHUNK, NVCHUNK)], vidx_v)

    for i in range(SEG_PER_TILE // 16):
      stage1d_v[pl.ds(i * 16, 16)] = zf

    def zrow(i, carry):
      stage2d_v[i, pl.ds(0, 16)] = zf
      stage2d_v[i, pl.ds(16, 16)] = zf
      return carry
    lax.fori_loop(0, 256, zrow, 0)

    for i in range(CHUNK // 16):
      ones_v[pl.ds(i * 16, 16)] = onef

    pltpu.sync_copy(stage1d_v, cnt_s.at[pl.ds(s * SEG_PER_TILE, SEG_PER_TILE)])
    for k in range(SEG_PER_TILE // 256):
      pltpu.sync_copy(stage2d_v,
                      acc_s.at[pl.ds(s * SEG_PER_TILE + k * 256, 256)])

    lane = lax.iota(jnp.int32, 16)

    def seg_chunk(j, carry):
      base = wid * POS_PER_W + j * CHUNK
      for k in range(CHUNK // 16):
        pos = base + k * 16 + lane
        lo = jnp.zeros((16,), jnp.int32)
        sz = B // 2
        while sz >= 1:
          cand = lo + sz
          oc = plsc.load_gather(off_v, [cand])
          lo = jnp.where(oc <= pos, cand, lo)
          sz //= 2
        seg_v[j, pl.ds(k * 16, 16)] = lo
      return carry
    lax.fori_loop(0, NCHUNK, seg_chunk, 0)

    plsc.subcore_barrier()

    def bag_chunk(j, carry):
      pltpu.async_copy(uw_hbm.at[idx_v.at[j]], rows_v, sem).wait()
      pltpu.sync_copy(rows_v, acc_s.at[seg_v.at[j]], add=True)
      pltpu.sync_copy(ones_v, cnt_s.at[seg_v.at[j]], add=True)
      return carry
    lax.fori_loop(0, NCHUNK, bag_chunk, 0)

    def v_chunk(j, carry):
      pltpu.async_copy(vw_hbm.at[vidx_v.at[j]], vrows_v, sem).wait()
      pltpu.sync_copy(vrows_v, embv_hbm.at[pl.ds(wid * V_PER_W + j * CHUNK,
                                                 CHUNK)])
      return carry
    lax.fori_loop(0, NVCHUNK, v_chunk, 0)

    plsc.subcore_barrier()

    pltpu.sync_copy(cnt_s.at[pl.ds(s * SEG_PER_TILE, SEG_PER_TILE)], stage1d_v)
    pltpu.sync_copy(stage1d_v, pcnt_hbm.at[c, pl.ds(s * SEG_PER_TILE,
                                                    SEG_PER_TILE)])
    for k in range(SEG_PER_TILE // 256):
      off0 = s * SEG_PER_TILE + k * 256
      pltpu.sync_copy(acc_s.at[pl.ds(off0, 256)], stage2d_v)
      pltpu.sync_copy(stage2d_v, psum_hbm.at[c, pl.ds(off0, 256)])

  return body(bag2d, offsets, v2d, u_weight, v_weight)


TBLK = 8192


TT = TBLK // 4


def _tr_body(in_ref, out_ref):
  x = in_ref[...]
  out_ref[...] = jnp.concatenate(
      [x[:, g * TT:(g + 1) * TT].T for g in range(4)], axis=1)


def _tc_transpose(table_t):
  V = table_t.shape[1]
  nblk = (V + TBLK - 1) // TBLK
  return pl.pallas_call(
      _tr_body,
      grid=(nblk,),
      in_specs=[pl.BlockSpec((D, TBLK), lambda i: (0, i))],
      out_specs=pl.BlockSpec((TT, 128), lambda i: (i, 0)),
      out_shape=jax.ShapeDtypeStruct((nblk * TT, 128), jnp.float32),
  )(table_t)


def _perm(r):
  return ((r >> 13) << 13) + ((r & (TT - 1)) << 2) + ((r >> 11) & 3)


BLK = 1024


def _tc_body(ps_ref, pc_ref, ev_ref, out_ref):
  ps = ps_ref[...]
  pc = pc_ref[...]
  ev = ev_ref[...]
  sums = ps[0] + ps[1]
  cnt = pc[0] + pc[1]
  emb_u = sums / jnp.maximum(cnt, 1.0)[:, None]
  out_ref[...] = jnp.sum(emb_u[:, None, :] * ev, axis=-1)


def _tc_dot(psum, pcnt, embv3):
  nblk = B // BLK
  return pl.pallas_call(
      _tc_body,
      grid=(nblk,),
      in_specs=[
          pl.BlockSpec((NC, BLK, D), lambda i: (0, i, 0)),
          pl.BlockSpec((NC, BLK), lambda i: (0, i)),
          pl.BlockSpec((BLK, NSAMP, D), lambda i: (i, 0, 0)),
      ],
      out_specs=pl.BlockSpec((BLK, NSAMP), lambda i: (i, 0)),
      out_shape=jax.ShapeDtypeStruct((B, NSAMP), jnp.float32),
  )(psum, pcnt, embv3)


@jax.jit
def kernel(bag, offsets, v, u_weight, v_weight):
  bag2d = _perm(bag.astype(jnp.int32)).reshape(NW * NCHUNK, CHUNK)
  v2d = _perm(v.astype(jnp.int32)).reshape(NW * NVCHUNK, CHUNK)
  uw = _tc_transpose(u_weight.T)
  vw = _tc_transpose(v_weight.T)
  uw = uw.reshape(uw.shape[0] * 4, D)
  vw = vw.reshape(vw.shape[0] * 4, D)
  psum, pcnt, embv = _sc_kernel(bag2d, offsets.astype(jnp.int32), v2d,
                                uw, vw)
  return _tc_dot(psum, pcnt, embv.reshape(B, NSAMP, D))


# speedup vs baseline: 1.9646x; 1.0396x over previous
import functools

import jax
import jax.numpy as jnp
from jax import lax
from jax.experimental import pallas as pl
from jax.experimental.pallas import tpu as pltpu
from jax.experimental.pallas import tpu_sc as plsc

B = 16384
D = 32
TOTAL = 131072
NSAMP = 6
NC, NS = 2, 16
NW = NC * NS
CHUNK = 128
POS_PER_W = TOTAL // NW
NCHUNK = POS_PER_W // CHUNK
VTOT = B * NSAMP
V_PER_W = VTOT // NW
NVCHUNK = V_PER_W // CHUNK
SEG_PER_TILE = B // NS


def _sc_kernel(bag2d, offsets, v2d, vdst2d, u_weight, v_weight):
  mesh = plsc.VectorSubcoreMesh(core_axis_name="c", subcore_axis_name="s")

  @functools.partial(
      pl.kernel,
      out_type=(
          jax.ShapeDtypeStruct((NC, B, D), jnp.float32),
          jax.ShapeDtypeStruct((NC, B), jnp.float32),
          jax.ShapeDtypeStruct((VTOT, D), jnp.float32),
      ),
      mesh=mesh,
      compiler_params=pltpu.CompilerParams(needs_layout_passes=False,
                                           use_tc_tiling_on_sc=False),
      scratch_types=(
          pltpu.VMEM((B,), jnp.int32),
          pltpu.VMEM((NCHUNK, CHUNK), jnp.int32),
          pltpu.VMEM((NCHUNK, CHUNK), jnp.int32),
          pltpu.VMEM((NVCHUNK, CHUNK), jnp.int32),
          pltpu.VMEM((NVCHUNK, CHUNK), jnp.int32),
          pltpu.VMEM((CHUNK, D), jnp.float32),
          pltpu.VMEM((CHUNK, D), jnp.float32),
          pltpu.VMEM((CHUNK,), jnp.float32),
          pltpu.VMEM((256, D), jnp.float32),
          pltpu.VMEM((SEG_PER_TILE,), jnp.float32),
          pltpu.VMEM_SHARED((B, D), jnp.float32),
          pltpu.VMEM_SHARED((B,), jnp.float32),
          pltpu.SemaphoreType.DMA,
      ),
  )
  def body(bag_hbm, off_hbm, v_hbm, vdst_hbm, uw_hbm, vw_hbm,
           psum_hbm, pcnt_hbm, embv_hbm,
           off_v, idx_v, seg_v, vidx_v, vdst_v, rows_v, vrows_v, ones_v,
           stage2d_v, stage1d_v, acc_s, cnt_s, sem):
    c = lax.axis_index("c")
    s = lax.axis_index("s")
    wid = s * NC + c

    zf = jnp.zeros((16,), jnp.float32)
    onef = jnp.full((16,), 1.0, jnp.float32)

    pltpu.sync_copy(off_hbm, off_v)
    pltpu.sync_copy(bag_hbm.at[pl.ds(wid * NCHUNK, NCHUNK)], idx_v)
    pltpu.sync_copy(v_hbm.at[pl.ds(wid * NVCHUNK, NVCHUNK)], vidx_v)
    pltpu.sync_copy(vdst_hbm.at[pl.ds(wid * NVCHUNK, NVCHUNK)], vdst_v)

    for i in range(SEG_PER_TILE // 16):
      stage1d_v[pl.ds(i * 16, 16)] = zf

    def zrow(i, carry):
      stage2d_v[i, pl.ds(0, 16)] = zf
      stage2d_v[i, pl.ds(16, 16)] = zf
      return carry
    lax.fori_loop(0, 256, zrow, 0)

    for i in range(CHUNK // 16):
      ones_v[pl.ds(i * 16, 16)] = onef

    pltpu.sync_copy(stage1d_v, cnt_s.at[pl.ds(s * SEG_PER_TILE, SEG_PER_TILE)])
    for k in range(SEG_PER_TILE // 256):
      pltpu.sync_copy(stage2d_v,
                      acc_s.at[pl.ds(s * SEG_PER_TILE + k * 256, 256)])

    lane = lax.iota(jnp.int32, 16)

    def seg_chunk(j, carry):
      base = wid * POS_PER_W + j * CHUNK
      for k in range(CHUNK // 16):
        pos = base + k * 16 + lane
        lo = jnp.zeros((16,), jnp.int32)
        sz = B // 2
        while sz >= 1:
          cand = lo + sz
          oc = plsc.load_gather(off_v, [cand])
          lo = jnp.where(oc <= pos, cand, lo)
          sz //= 2
        seg_v[j, pl.ds(k * 16, 16)] = lo
      return carry
    lax.fori_loop(0, NCHUNK, seg_chunk, 0)

    plsc.subcore_barrier()

    def bag_chunk(j, carry):
      pltpu.async_copy(uw_hbm.at[idx_v.at[j]], rows_v, sem).wait()
      pltpu.sync_copy(rows_v, acc_s.at[seg_v.at[j]], add=True)
      pltpu.sync_copy(ones_v, cnt_s.at[seg_v.at[j]], add=True)
      return carry
    lax.fori_loop(0, NCHUNK, bag_chunk, 0)

    def v_chunk(j, carry):
      pltpu.async_copy(vw_hbm.at[vidx_v.at[j]], vrows_v, sem).wait()
      pltpu.async_copy(vrows_v, embv_hbm.at[vdst_v.at[j]], sem).wait()
      return carry
    lax.fori_loop(0, NVCHUNK, v_chunk, 0)

    plsc.subcore_barrier()

    pltpu.sync_copy(cnt_s.at[pl.ds(s * SEG_PER_TILE, SEG_PER_TILE)], stage1d_v)
    pltpu.sync_copy(stage1d_v, pcnt_hbm.at[c, pl.ds(s * SEG_PER_TILE,
                                                    SEG_PER_TILE)])
    for k in range(SEG_PER_TILE // 256):
      off0 = s * SEG_PER_TILE + k * 256
      pltpu.sync_copy(acc_s.at[pl.ds(off0, 256)], stage2d_v)
      pltpu.sync_copy(stage2d_v, psum_hbm.at[c, pl.ds(off0, 256)])

  return body(bag2d, offsets, v2d, vdst2d, u_weight, v_weight)


TBLK = 16384
TT = TBLK // 4
_SH = TT.bit_length() - 1


def _tr_body(in_ref, out_ref):
  x = in_ref[...]
  out_ref[...] = jnp.concatenate(
      [x[:, g * TT:(g + 1) * TT].T for g in range(4)], axis=1)


def _tc_transpose(table_t):
  V = table_t.shape[1]
  nblk = (V + TBLK - 1) // TBLK
  return pl.pallas_call(
      _tr_body,
      grid=(nblk,),
      in_specs=[pl.BlockSpec((D, TBLK), lambda i: (0, i))],
      out_specs=pl.BlockSpec((TT, 128), lambda i: (i, 0)),
      out_shape=jax.ShapeDtypeStruct((nblk * TT, 128), jnp.float32),
  )(table_t)


def _perm(r):
  return (((r >> (_SH + 2)) << (_SH + 2)) + ((r & (TT - 1)) << 2)
          + ((r >> _SH) & 3))


BLK = 1024


def _tc_body(ps_ref, pc_ref, ev_ref, out_ref):
  ps = ps_ref[...]
  pc = pc_ref[...]
  ev = ev_ref[...]
  sums = ps[0] + ps[1]
  cnt = pc[0] + pc[1]
  emb_u = sums / jnp.maximum(cnt, 1.0)[:, None]
  out_ref[...] = jnp.sum(emb_u[None, :, :] * ev, axis=-1)


def _tc_dot(psum, pcnt, embv_g):
  nblk = B // BLK
  return pl.pallas_call(
      _tc_body,
      grid=(nblk,),
      in_specs=[
          pl.BlockSpec((NC, BLK, D), lambda i: (0, i, 0)),
          pl.BlockSpec((NC, BLK), lambda i: (0, i)),
          pl.BlockSpec((NSAMP, BLK, D), lambda i: (0, i, 0)),
      ],
      out_specs=pl.BlockSpec((NSAMP, BLK), lambda i: (0, i)),
      out_shape=jax.ShapeDtypeStruct((NSAMP, B), jnp.float32),
  )(psum, pcnt, embv_g)


@jax.jit
def kernel(bag, offsets, v, u_weight, v_weight):
  bag2d = _perm(bag.astype(jnp.int32)).reshape(NW * NCHUNK, CHUNK)
  v2d = _perm(v.astype(jnp.int32)).reshape(NW * NVCHUNK, CHUNK)
  t = jnp.arange(VTOT, dtype=jnp.int32)
  vdst2d = ((t % NSAMP) * B + t // NSAMP).reshape(NW * NVCHUNK, CHUNK)
  uw = _tc_transpose(u_weight.T)
  vw = _tc_transpose(v_weight.T)
  uw = uw.reshape(uw.shape[0] * 4, D)
  vw = vw.reshape(vw.shape[0] * 4, D)
  psum, pcnt, embv = _sc_kernel(bag2d, offsets.astype(jnp.int32), v2d,
                                vdst2d, uw, vw)
  s6 = _tc_dot(psum, pcnt, embv.reshape(NSAMP, B, D))
  return s6.T


# speedup vs baseline: 3.3662x; 1.7134x over previous
import functools

import jax
import jax.numpy as jnp
from jax import lax
from jax.experimental import pallas as pl
from jax.experimental.pallas import tpu as pltpu
from jax.experimental.pallas import tpu_sc as plsc

B = 16384
D = 32
TOTAL = 131072
NSAMP = 6
NC, NS = 2, 16
NW = NC * NS
CHUNK = 128
POS_PER_W = TOTAL // NW
NCHUNK = POS_PER_W // CHUNK
VTOT = B * NSAMP
V_PER_W = VTOT // NW
NVCHUNK = V_PER_W // CHUNK
SEG_PER_TILE = B // NS


def _sc_kernel(bag2d, offsets, v2d, vdst2d, u_weight, v_weight):
  mesh = plsc.VectorSubcoreMesh(core_axis_name="c", subcore_axis_name="s")

  @functools.partial(
      pl.kernel,
      out_type=(
          jax.ShapeDtypeStruct((NC, B, D), jnp.float32),
          jax.ShapeDtypeStruct((NC, B), jnp.float32),
          jax.ShapeDtypeStruct((VTOT, D), jnp.float32),
      ),
      mesh=mesh,
      compiler_params=pltpu.CompilerParams(needs_layout_passes=False,
                                           use_tc_tiling_on_sc=False),
      scratch_types=(
          pltpu.VMEM((B,), jnp.int32),
          pltpu.VMEM((NCHUNK, CHUNK), jnp.int32),
          pltpu.VMEM((NCHUNK, CHUNK), jnp.int32),
          pltpu.VMEM((NVCHUNK, CHUNK), jnp.int32),
          pltpu.VMEM((NVCHUNK, CHUNK), jnp.int32),
          pltpu.VMEM((CHUNK, D), jnp.float32),
          pltpu.VMEM((CHUNK, D), jnp.float32),
          pltpu.VMEM((CHUNK,), jnp.float32),
          pltpu.VMEM((256, D), jnp.float32),
          pltpu.VMEM((SEG_PER_TILE,), jnp.float32),
          pltpu.VMEM_SHARED((B, D), jnp.float32),
          pltpu.VMEM_SHARED((B,), jnp.float32),
          pltpu.SemaphoreType.DMA,
      ),
  )
  def body(bag_hbm, off_hbm, v_hbm, vdst_hbm, uw_hbm, vw_hbm,
           psum_hbm, pcnt_hbm, embv_hbm,
           off_v, idx_v, seg_v, vidx_v, vdst_v, rows_v, vrows_v, ones_v,
           stage2d_v, stage1d_v, acc_s, cnt_s, sem):
    c = lax.axis_index("c")
    s = lax.axis_index("s")
    wid = s * NC + c

    zf = jnp.zeros((16,), jnp.float32)
    onef = jnp.full((16,), 1.0, jnp.float32)

    pltpu.sync_copy(off_hbm, off_v)
    pltpu.sync_copy(bag_hbm.at[pl.ds(wid * NCHUNK, NCHUNK)], idx_v)
    pltpu.sync_copy(v_hbm.at[pl.ds(wid * NVCHUNK, NVCHUNK)], vidx_v)
    pltpu.sync_copy(vdst_hbm.at[pl.ds(wid * NVCHUNK, NVCHUNK)], vdst_v)

    for i in range(SEG_PER_TILE // 16):
      stage1d_v[pl.ds(i * 16, 16)] = zf

    def zrow(i, carry):
      stage2d_v[i, pl.ds(0, 16)] = zf
      stage2d_v[i, pl.ds(16, 16)] = zf
      return carry
    lax.fori_loop(0, 256, zrow, 0)

    for i in range(CHUNK // 16):
      ones_v[pl.ds(i * 16, 16)] = onef

    pltpu.sync_copy(stage1d_v, cnt_s.at[pl.ds(s * SEG_PER_TILE, SEG_PER_TILE)])
    for k in range(SEG_PER_TILE // 256):
      pltpu.sync_copy(stage2d_v,
                      acc_s.at[pl.ds(s * SEG_PER_TILE + k * 256, 256)])

    lane = lax.iota(jnp.int32, 16)

    def seg_chunk(j, carry):
      base = wid * POS_PER_W + j * CHUNK
      for k in range(CHUNK // 16):
        pos = base + k * 16 + lane
        lo = jnp.zeros((16,), jnp.int32)
        sz = B // 2
        while sz >= 1:
          cand = lo + sz
          oc = plsc.load_gather(off_v, [cand])
          lo = jnp.where(oc <= pos, cand, lo)
          sz //= 2
        seg_v[j, pl.ds(k * 16, 16)] = lo
      return carry
    lax.fori_loop(0, NCHUNK, seg_chunk, 0)

    plsc.subcore_barrier()

    def bag_chunk(j, carry):
      pltpu.async_copy(uw_hbm.at[idx_v.at[j]], rows_v, sem).wait()
      pltpu.sync_copy(rows_v, acc_s.at[seg_v.at[j]], add=True)
      pltpu.sync_copy(ones_v, cnt_s.at[seg_v.at[j]], add=True)
      return carry
    lax.fori_loop(0, NCHUNK, bag_chunk, 0)

    def v_chunk(j, carry):
      pltpu.async_copy(vw_hbm.at[vidx_v.at[j]], vrows_v, sem).wait()
      pltpu.async_copy(vrows_v, embv_hbm.at[vdst_v.at[j]], sem).wait()
      return carry
    lax.fori_loop(0, NVCHUNK, v_chunk, 0)

    plsc.subcore_barrier()

    pltpu.sync_copy(cnt_s.at[pl.ds(s * SEG_PER_TILE, SEG_PER_TILE)], stage1d_v)
    pltpu.sync_copy(stage1d_v, pcnt_hbm.at[c, pl.ds(s * SEG_PER_TILE,
                                                    SEG_PER_TILE)])
    for k in range(SEG_PER_TILE // 256):
      off0 = s * SEG_PER_TILE + k * 256
      pltpu.sync_copy(acc_s.at[pl.ds(off0, 256)], stage2d_v)
      pltpu.sync_copy(stage2d_v, psum_hbm.at[c, pl.ds(off0, 256)])

  return body(bag2d, offsets, v2d, vdst2d, u_weight, v_weight)


TBLK = 16384
TT = TBLK // 4
_SH = TT.bit_length() - 1


def _tr_body(in_ref, out_ref):
  x = in_ref[...]
  out_ref[...] = jnp.concatenate(
      [x[:, g * TT:(g + 1) * TT] for g in range(4)], axis=0).T


def _tc_transpose(table_t):
  V = table_t.shape[1]
  nblk = (V + TBLK - 1) // TBLK
  return pl.pallas_call(
      _tr_body,
      grid=(nblk,),
      in_specs=[pl.BlockSpec((D, TBLK), lambda i: (0, i))],
      out_specs=pl.BlockSpec((TT, 128), lambda i: (i, 0)),
      out_shape=jax.ShapeDtypeStruct((nblk * TT, 128), jnp.float32),
  )(table_t)


def _perm(r):
  return (((r >> (_SH + 2)) << (_SH + 2)) + ((r & (TT - 1)) << 2)
          + ((r >> _SH) & 3))


BLK = 1024


def _tc_body(ps_ref, pc_ref, ev_ref, out_ref):
  ps = ps_ref[...]
  pc = pc_ref[...]
  ev = ev_ref[...]
  sums = ps[0] + ps[1]
  cnt = pc[0] + pc[1]
  emb_u = sums / jnp.maximum(cnt, 1.0)[:, None]
  out_ref[...] = jnp.sum(emb_u[None, :, :] * ev, axis=-1)


def _tc_dot(psum, pcnt, embv_g):
  nblk = B // BLK
  return pl.pallas_call(
      _tc_body,
      grid=(nblk,),
      in_specs=[
          pl.BlockSpec((NC, BLK, D), lambda i: (0, i, 0)),
          pl.BlockSpec((NC, BLK), lambda i: (0, i)),
          pl.BlockSpec((NSAMP, BLK, D), lambda i: (0, i, 0)),
      ],
      out_specs=pl.BlockSpec((NSAMP, BLK), lambda i: (0, i)),
      out_shape=jax.ShapeDtypeStruct((NSAMP, B), jnp.float32),
  )(psum, pcnt, embv_g)


@jax.jit
def kernel(bag, offsets, v, u_weight, v_weight):
  bag2d = _perm(bag.astype(jnp.int32)).reshape(NW * NCHUNK, CHUNK)
  v2d = _perm(v.astype(jnp.int32)).reshape(NW * NVCHUNK, CHUNK)
  t = jnp.arange(VTOT, dtype=jnp.int32)
  vdst2d = ((t % NSAMP) * B + t // NSAMP).reshape(NW * NVCHUNK, CHUNK)
  uw = _tc_transpose(u_weight.T)
  vw = _tc_transpose(v_weight.T)
  uw = uw.reshape(uw.shape[0] * 4, D)
  vw = vw.reshape(vw.shape[0] * 4, D)
  psum, pcnt, embv = _sc_kernel(bag2d, offsets.astype(jnp.int32), v2d,
                                vdst2d, uw, vw)
  s6 = _tc_dot(psum, pcnt, embv.reshape(NSAMP, B, D))
  return s6.T


# speedup vs baseline: 4.1667x; 1.2378x over previous
import functools

import jax
import jax.numpy as jnp
from jax import lax
from jax.experimental import pallas as pl
from jax.experimental.pallas import tpu as pltpu
from jax.experimental.pallas import tpu_sc as plsc

B = 16384
D = 32
TOTAL = 131072
NSAMP = 6
NC, NS = 2, 16
NW = NC * NS
CHUNK = 128
POS_PER_W = TOTAL // NW
NCHUNK = POS_PER_W // CHUNK
VTOT = B * NSAMP
V_PER_W = VTOT // NW
NVCHUNK = V_PER_W // CHUNK
SEG_PER_TILE = B // NS


def _sc_u_kernel(bag2d, offsets, u_weight):
  mesh = plsc.VectorSubcoreMesh(core_axis_name="c", subcore_axis_name="s")

  @functools.partial(
      pl.kernel,
      out_type=(
          jax.ShapeDtypeStruct((NC, B, D), jnp.float32),
          jax.ShapeDtypeStruct((NC, B), jnp.float32),
      ),
      mesh=mesh,
      compiler_params=pltpu.CompilerParams(needs_layout_passes=False,
                                           use_tc_tiling_on_sc=False),
      scratch_types=(
          pltpu.VMEM((B,), jnp.int32),
          pltpu.VMEM((NCHUNK, CHUNK), jnp.int32),
          pltpu.VMEM((NCHUNK, CHUNK), jnp.int32),
          pltpu.VMEM((CHUNK, D), jnp.float32),
          pltpu.VMEM((CHUNK,), jnp.float32),
          pltpu.VMEM((256, D), jnp.float32),
          pltpu.VMEM((SEG_PER_TILE,), jnp.float32),
          pltpu.VMEM_SHARED((B, D), jnp.float32),
          pltpu.VMEM_SHARED((B,), jnp.float32),
          pltpu.SemaphoreType.DMA,
      ),
  )
  def body(bag_hbm, off_hbm, uw_hbm,
           psum_hbm, pcnt_hbm,
           off_v, idx_v, seg_v, rows_v, ones_v,
           stage2d_v, stage1d_v, acc_s, cnt_s, sem):
    c = lax.axis_index("c")
    s = lax.axis_index("s")
    wid = s * NC + c

    zf = jnp.zeros((16,), jnp.float32)
    onef = jnp.full((16,), 1.0, jnp.float32)

    pltpu.sync_copy(off_hbm, off_v)
    pltpu.sync_copy(bag_hbm.at[pl.ds(wid * NCHUNK, NCHUNK)], idx_v)

    for i in range(SEG_PER_TILE // 16):
      stage1d_v[pl.ds(i * 16, 16)] = zf

    def zrow(i, carry):
      stage2d_v[i, pl.ds(0, 16)] = zf
      stage2d_v[i, pl.ds(16, 16)] = zf
      return carry
    lax.fori_loop(0, 256, zrow, 0)

    for i in range(CHUNK // 16):
      ones_v[pl.ds(i * 16, 16)] = onef

    pltpu.sync_copy(stage1d_v, cnt_s.at[pl.ds(s * SEG_PER_TILE, SEG_PER_TILE)])
    for k in range(SEG_PER_TILE // 256):
      pltpu.sync_copy(stage2d_v,
                      acc_s.at[pl.ds(s * SEG_PER_TILE + k * 256, 256)])

    lane = lax.iota(jnp.int32, 16)

    def seg_chunk(j, carry):
      base = wid * POS_PER_W + j * CHUNK
      for k in range(CHUNK // 16):
        pos = base + k * 16 + lane
        lo = jnp.zeros((16,), jnp.int32)
        sz = B // 2
        while sz >= 1:
          cand = lo + sz
          oc = plsc.load_gather(off_v, [cand])
          lo = jnp.where(oc <= pos, cand, lo)
          sz //= 2
        seg_v[j, pl.ds(k * 16, 16)] = lo
      return carry
    lax.fori_loop(0, NCHUNK, seg_chunk, 0)

    plsc.subcore_barrier()

    def bag_chunk(j, carry):
      pltpu.async_copy(uw_hbm.at[idx_v.at[j]], rows_v, sem).wait()
      pltpu.sync_copy(rows_v, acc_s.at[seg_v.at[j]], add=True)
      pltpu.sync_copy(ones_v, cnt_s.at[seg_v.at[j]], add=True)
      return carry
    lax.fori_loop(0, NCHUNK, bag_chunk, 0)

    plsc.subcore_barrier()

    pltpu.sync_copy(cnt_s.at[pl.ds(s * SEG_PER_TILE, SEG_PER_TILE)], stage1d_v)
    pltpu.sync_copy(stage1d_v, pcnt_hbm.at[c, pl.ds(s * SEG_PER_TILE,
                                                    SEG_PER_TILE)])
    for k in range(SEG_PER_TILE // 256):
      off0 = s * SEG_PER_TILE + k * 256
      pltpu.sync_copy(acc_s.at[pl.ds(off0, 256)], stage2d_v)
      pltpu.sync_copy(stage2d_v, psum_hbm.at[c, pl.ds(off0, 256)])

  return body(bag2d, offsets, u_weight)


def _sc_v_kernel(v2d, vdst2d, v_weight):
  mesh = plsc.VectorSubcoreMesh(core_axis_name="c", subcore_axis_name="s")

  @functools.partial(
      pl.kernel,
      out_type=jax.ShapeDtypeStruct((VTOT, D), jnp.float32),
      mesh=mesh,
      compiler_params=pltpu.CompilerParams(needs_layout_passes=False,
                                           use_tc_tiling_on_sc=False),
      scratch_types=(
          pltpu.VMEM((NVCHUNK, CHUNK), jnp.int32),
          pltpu.VMEM((NVCHUNK, CHUNK), jnp.int32),
          pltpu.VMEM((CHUNK, D), jnp.float32),
          pltpu.SemaphoreType.DMA,
      ),
  )
  def body(v_hbm, vdst_hbm, vw_hbm, embv_hbm, vidx_v, vdst_v, vrows_v, sem):
    c = lax.axis_index("c")
    s = lax.axis_index("s")
    wid = s * NC + c
    pltpu.sync_copy(v_hbm.at[pl.ds(wid * NVCHUNK, NVCHUNK)], vidx_v)
    pltpu.sync_copy(vdst_hbm.at[pl.ds(wid * NVCHUNK, NVCHUNK)], vdst_v)

    def v_chunk(j, carry):
      pltpu.async_copy(vw_hbm.at[vidx_v.at[j]], vrows_v, sem).wait()
      pltpu.async_copy(vrows_v, embv_hbm.at[vdst_v.at[j]], sem).wait()
      return carry
    lax.fori_loop(0, NVCHUNK, v_chunk, 0)

  return body(v2d, vdst2d, v_weight)


TBLK = 16384
TT = TBLK // 4
_SH = TT.bit_length() - 1


def _tr_body(in_ref, out_ref):
  x = in_ref[...]
  out_ref[...] = jnp.concatenate(
      [x[:, g * TT:(g + 1) * TT] for g in range(4)], axis=0).T


def _tc_transpose(table_t):
  V = table_t.shape[1]
  nblk = (V + TBLK - 1) // TBLK
  return pl.pallas_call(
      _tr_body,
      grid=(nblk,),
      in_specs=[pl.BlockSpec((D, TBLK), lambda i: (0, i))],
      out_specs=pl.BlockSpec((TT, 128), lambda i: (i, 0)),
      out_shape=jax.ShapeDtypeStruct((nblk * TT, 128), jnp.float32),
  )(table_t)


def _perm(r):
  return (((r >> (_SH + 2)) << (_SH + 2)) + ((r & (TT - 1)) << 2)
          + ((r >> _SH) & 3))


BLK = 1024


def _tc_body(ps_ref, pc_ref, ev_ref, out_ref):
  ps = ps_ref[...]
  pc = pc_ref[...]
  ev = ev_ref[...]
  sums = ps[0] + ps[1]
  cnt = pc[0] + pc[1]
  emb_u = sums / jnp.maximum(cnt, 1.0)[:, None]
  out_ref[...] = jnp.sum(emb_u[None, :, :] * ev, axis=-1)


def _tc_dot(psum, pcnt, embv_g):
  nblk = B // BLK
  return pl.pallas_call(
      _tc_body,
      grid=(nblk,),
      in_specs=[
          pl.BlockSpec((NC, BLK, D), lambda i: (0, i, 0)),
          pl.BlockSpec((NC, BLK), lambda i: (0, i)),
          pl.BlockSpec((NSAMP, BLK, D), lambda i: (0, i, 0)),
      ],
      out_specs=pl.BlockSpec((NSAMP, BLK), lambda i: (0, i)),
      out_shape=jax.ShapeDtypeStruct((NSAMP, B), jnp.float32),
  )(psum, pcnt, embv_g)


@jax.jit
def kernel(bag, offsets, v, u_weight, v_weight):
  bag2d = _perm(bag.astype(jnp.int32)).reshape(NW * NCHUNK, CHUNK)
  v2d = _perm(v.astype(jnp.int32)).reshape(NW * NVCHUNK, CHUNK)
  t = jnp.arange(VTOT, dtype=jnp.int32)
  vdst2d = ((t % NSAMP) * B + t // NSAMP).reshape(NW * NVCHUNK, CHUNK)
  uw = _tc_transpose(u_weight.T)
  uw = uw.reshape(uw.shape[0] * 4, D)
  psum, pcnt = _sc_u_kernel(bag2d, offsets.astype(jnp.int32), uw)
  vw = _tc_transpose(v_weight.T)
  vw = vw.reshape(vw.shape[0] * 4, D)
  embv = _sc_v_kernel(v2d, vdst2d, vw)
  s6 = _tc_dot(psum, pcnt, embv.reshape(NSAMP, B, D))
  return s6.T


# speedup vs baseline: 4.8406x; 1.1617x over previous
import functools

import jax
import jax.numpy as jnp
from jax import lax
from jax.experimental import pallas as pl
from jax.experimental.pallas import tpu as pltpu
from jax.experimental.pallas import tpu_sc as plsc

B = 16384
D = 32
TOTAL = 131072
NSAMP = 6
NC, NS = 2, 16
NW = NC * NS
CHUNK = 128
POS_PER_W = TOTAL // NW
NCHUNK = POS_PER_W // CHUNK
VTOT = B * NSAMP
V_PER_W = VTOT // NW
NVCHUNK = V_PER_W // CHUNK
SEG_PER_TILE = B // NS


def _sc_u_kernel(bag2d, offsets, u_weight):
  mesh = plsc.VectorSubcoreMesh(core_axis_name="c", subcore_axis_name="s")

  @functools.partial(
      pl.kernel,
      out_type=(
          jax.ShapeDtypeStruct((NC, B, D), jnp.float32),
          jax.ShapeDtypeStruct((NC, B), jnp.float32),
      ),
      mesh=mesh,
      compiler_params=pltpu.CompilerParams(needs_layout_passes=False,
                                           use_tc_tiling_on_sc=False),
      scratch_types=(
          pltpu.VMEM((B,), jnp.int32),
          pltpu.VMEM((NCHUNK, CHUNK), jnp.int32),
          pltpu.VMEM((NCHUNK, CHUNK), jnp.int32),
          pltpu.VMEM((CHUNK, D), jnp.float32),
          pltpu.VMEM((CHUNK,), jnp.float32),
          pltpu.VMEM((256, D), jnp.float32),
          pltpu.VMEM((SEG_PER_TILE,), jnp.float32),
          pltpu.VMEM_SHARED((B, D), jnp.float32),
          pltpu.VMEM_SHARED((B,), jnp.float32),
          pltpu.SemaphoreType.DMA,
      ),
  )
  def body(bag_hbm, off_hbm, uw_hbm,
           psum_hbm, pcnt_hbm,
           off_v, idx_v, seg_v, rows_v, ones_v,
           stage2d_v, stage1d_v, acc_s, cnt_s, sem):
    c = lax.axis_index("c")
    s = lax.axis_index("s")
    wid = s * NC + c

    zf = jnp.zeros((16,), jnp.float32)
    onef = jnp.full((16,), 1.0, jnp.float32)

    pltpu.sync_copy(off_hbm, off_v)
    pltpu.sync_copy(bag_hbm.at[pl.ds(wid * NCHUNK, NCHUNK)], idx_v)

    for i in range(SEG_PER_TILE // 16):
      stage1d_v[pl.ds(i * 16, 16)] = zf

    def zrow(i, carry):
      stage2d_v[i, pl.ds(0, 16)] = zf
      stage2d_v[i, pl.ds(16, 16)] = zf
      return carry
    lax.fori_loop(0, 256, zrow, 0)

    for i in range(CHUNK // 16):
      ones_v[pl.ds(i * 16, 16)] = onef

    pltpu.sync_copy(stage1d_v, cnt_s.at[pl.ds(s * SEG_PER_TILE, SEG_PER_TILE)])
    for k in range(SEG_PER_TILE // 256):
      pltpu.sync_copy(stage2d_v,
                      acc_s.at[pl.ds(s * SEG_PER_TILE + k * 256, 256)])

    lane = lax.iota(jnp.int32, 16)

    def seg_chunk(j, carry):
      base = wid * POS_PER_W + j * CHUNK
      for k in range(CHUNK // 16):
        pos = base + k * 16 + lane
        lo = jnp.zeros((16,), jnp.int32)
        sz = B // 2
        while sz >= 1:
          cand = lo + sz
          oc = plsc.load_gather(off_v, [cand])
          lo = jnp.where(oc <= pos, cand, lo)
          sz //= 2
        seg_v[j, pl.ds(k * 16, 16)] = lo
      return carry
    lax.fori_loop(0, NCHUNK, seg_chunk, 0)

    plsc.subcore_barrier()

    def bag_chunk(j, carry):
      pltpu.async_copy(uw_hbm.at[idx_v.at[j]], rows_v, sem).wait()
      pltpu.sync_copy(rows_v, acc_s.at[seg_v.at[j]], add=True)
      pltpu.sync_copy(ones_v, cnt_s.at[seg_v.at[j]], add=True)
      return carry
    lax.fori_loop(0, NCHUNK, bag_chunk, 0)

    plsc.subcore_barrier()

    pltpu.sync_copy(cnt_s.at[pl.ds(s * SEG_PER_TILE, SEG_PER_TILE)], stage1d_v)
    pltpu.sync_copy(stage1d_v, pcnt_hbm.at[c, pl.ds(s * SEG_PER_TILE,
                                                    SEG_PER_TILE)])
    for k in range(SEG_PER_TILE // 256):
      off0 = s * SEG_PER_TILE + k * 256
      pltpu.sync_copy(acc_s.at[pl.ds(off0, 256)], stage2d_v)
      pltpu.sync_copy(stage2d_v, psum_hbm.at[c, pl.ds(off0, 256)])

  return body(bag2d, offsets, u_weight)


def _sc_v_kernel(v2d, vdst2d, v_weight):
  mesh = plsc.VectorSubcoreMesh(core_axis_name="c", subcore_axis_name="s")

  @functools.partial(
      pl.kernel,
      out_type=jax.ShapeDtypeStruct((VTOT * 4, D), jnp.float32),
      mesh=mesh,
      compiler_params=pltpu.CompilerParams(needs_layout_passes=False,
                                           use_tc_tiling_on_sc=False),
      scratch_types=(
          pltpu.VMEM((NVCHUNK, CHUNK), jnp.int32),
          pltpu.VMEM((NVCHUNK, CHUNK), jnp.int32),
          pltpu.VMEM((CHUNK, D), jnp.float32),
          pltpu.SemaphoreType.DMA,
      ),
  )
  def body(v_hbm, vdst_hbm, vw_hbm, embv_hbm, vidx_v, vdst_v, vrows_v, sem):
    c = lax.axis_index("c")
    s = lax.axis_index("s")
    wid = s * NC + c
    pltpu.sync_copy(v_hbm.at[pl.ds(wid * NVCHUNK, NVCHUNK)], vidx_v)
    pltpu.sync_copy(vdst_hbm.at[pl.ds(wid * NVCHUNK, NVCHUNK)], vdst_v)

    def v_chunk(j, carry):
      pltpu.async_copy(vw_hbm.at[vidx_v.at[j]], vrows_v, sem).wait()
      pltpu.async_copy(vrows_v, embv_hbm.at[vdst_v.at[j]], sem).wait()
      return carry
    lax.fori_loop(0, NVCHUNK, v_chunk, 0)

  return body(v2d, vdst2d, v_weight)


TBLK = 16384
TT = TBLK // 4
_SH = TT.bit_length() - 1


def _tr_body(in_ref, out_ref):
  x = in_ref[...]
  out_ref[...] = jnp.concatenate(
      [x[:, g * TT:(g + 1) * TT] for g in range(4)], axis=0).T


def _tc_transpose(table_t):
  V = table_t.shape[1]
  nblk = (V + TBLK - 1) // TBLK
  return pl.pallas_call(
      _tr_body,
      grid=(nblk,),
      in_specs=[pl.BlockSpec((D, TBLK), lambda i: (0, i))],
      out_specs=pl.BlockSpec((TT, 128), lambda i: (i, 0)),
      out_shape=jax.ShapeDtypeStruct((nblk * TT, 128), jnp.float32),
  )(table_t)


def _perm(r):
  return (((r >> (_SH + 2)) << (_SH + 2)) + ((r & (TT - 1)) << 2)
          + ((r >> _SH) & 3))


BLK = 1024


def _tc_body(ps_ref, pc_ref, ev_ref, out_ref):
  ps = ps_ref[...]
  pc = pc_ref[...]
  ev = ev_ref[...][:, :, :D]
  sums = ps[0] + ps[1]
  cnt = pc[0] + pc[1]
  emb_u = sums / jnp.maximum(cnt, 1.0)[:, None]
  out_ref[...] = jnp.sum(emb_u[None, :, :] * ev, axis=-1)


def _tc_dot(psum, pcnt, embv_g):
  nblk = B // BLK
  return pl.pallas_call(
      _tc_body,
      grid=(nblk,),
      in_specs=[
          pl.BlockSpec((NC, BLK, D), lambda i: (0, i, 0)),
          pl.BlockSpec((NC, BLK), lambda i: (0, i)),
          pl.BlockSpec((NSAMP, BLK, 128), lambda i: (0, i, 0)),
      ],
      out_specs=pl.BlockSpec((NSAMP, BLK), lambda i: (0, i)),
      out_shape=jax.ShapeDtypeStruct((NSAMP, B), jnp.float32),
  )(psum, pcnt, embv_g)


@jax.jit
def kernel(bag, offsets, v, u_weight, v_weight):
  bag2d = _perm(bag.astype(jnp.int32)).reshape(NW * NCHUNK, CHUNK)
  v2d = _perm(v.T.astype(jnp.int32)).reshape(NW * NVCHUNK, CHUNK)
  vdst2d = (jnp.arange(VTOT, dtype=jnp.int32) * 4).reshape(
      NW * NVCHUNK, CHUNK)
  uw = _tc_transpose(u_weight.T)
  uw = uw.reshape(uw.shape[0] * 4, D)
  psum, pcnt = _sc_u_kernel(bag2d, offsets.astype(jnp.int32), uw)
  vw = _tc_transpose(v_weight.T)
  vw = vw.reshape(vw.shape[0] * 4, D)
  embv4 = _sc_v_kernel(v2d, vdst2d, vw)
  s6 = _tc_dot(psum, pcnt, embv4.reshape(NSAMP, B, 128))
  return s6.T


# speedup vs baseline: 4.9089x; 1.0141x over previous
import functools

import jax
import jax.numpy as jnp
from jax import lax
from jax.experimental import pallas as pl
from jax.experimental.pallas import tpu as pltpu
from jax.experimental.pallas import tpu_sc as plsc

B = 16384
D = 32
TOTAL = 131072
NSAMP = 6
NC, NS = 2, 16
NW = NC * NS
CHUNK = 128
POS_PER_W = TOTAL // NW
NCHUNK = POS_PER_W // CHUNK
VTOT = B * NSAMP
V_PER_W = VTOT // NW
NVCHUNK = V_PER_W // CHUNK
SEG_PER_TILE = B // NS


def _sc_u_kernel(bag2d, offsets, u_weight):
  mesh = plsc.VectorSubcoreMesh(core_axis_name="c", subcore_axis_name="s")

  @functools.partial(
      pl.kernel,
      out_type=(
          jax.ShapeDtypeStruct((NC, B, D), jnp.float32),
          jax.ShapeDtypeStruct((NC, B), jnp.float32),
      ),
      mesh=mesh,
      compiler_params=pltpu.CompilerParams(needs_layout_passes=False,
                                           use_tc_tiling_on_sc=False),
      scratch_types=(
          pltpu.VMEM((B,), jnp.int32),
          pltpu.VMEM((NCHUNK, CHUNK), jnp.int32),
          pltpu.VMEM((NCHUNK, CHUNK), jnp.int32),
          pltpu.VMEM((CHUNK, D), jnp.float32),
          pltpu.VMEM((CHUNK,), jnp.float32),
          pltpu.VMEM((256, D), jnp.float32),
          pltpu.VMEM((SEG_PER_TILE,), jnp.float32),
          pltpu.VMEM_SHARED((B, D), jnp.float32),
          pltpu.VMEM_SHARED((B,), jnp.float32),
          pltpu.SemaphoreType.DMA,
      ),
  )
  def body(bag_hbm, off_hbm, uw_hbm,
           psum_hbm, pcnt_hbm,
           off_v, idx_v, seg_v, rows_v, ones_v,
           stage2d_v, stage1d_v, acc_s, cnt_s, sem):
    c = lax.axis_index("c")
    s = lax.axis_index("s")
    wid = s * NC + c

    zf = jnp.zeros((16,), jnp.float32)
    onef = jnp.full((16,), 1.0, jnp.float32)

    pltpu.sync_copy(off_hbm, off_v)
    pltpu.sync_copy(bag_hbm.at[pl.ds(wid * NCHUNK, NCHUNK)], idx_v)

    for i in range(SEG_PER_TILE // 16):
      stage1d_v[pl.ds(i * 16, 16)] = zf

    def zrow(i, carry):
      stage2d_v[i, pl.ds(0, 16)] = zf
      stage2d_v[i, pl.ds(16, 16)] = zf
      return carry
    lax.fori_loop(0, 256, zrow, 0)

    for i in range(CHUNK // 16):
      ones_v[pl.ds(i * 16, 16)] = onef

    pltpu.sync_copy(stage1d_v, cnt_s.at[pl.ds(s * SEG_PER_TILE, SEG_PER_TILE)])
    for k in range(SEG_PER_TILE // 256):
      pltpu.sync_copy(stage2d_v,
                      acc_s.at[pl.ds(s * SEG_PER_TILE + k * 256, 256)])

    lane = lax.iota(jnp.int32, 16)

    def seg_chunk(j, carry):
      base = wid * POS_PER_W + j * CHUNK
      for k in range(CHUNK // 16):
        pos = base + k * 16 + lane
        lo = jnp.zeros((16,), jnp.int32)
        sz = B // 2
        while sz >= 1:
          cand = lo + sz
          oc = plsc.load_gather(off_v, [cand])
          lo = jnp.where(oc <= pos, cand, lo)
          sz //= 2
        seg_v[j, pl.ds(k * 16, 16)] = lo
      return carry
    lax.fori_loop(0, NCHUNK, seg_chunk, 0)

    plsc.subcore_barrier()

    def bag_chunk(j, carry):
      pltpu.async_copy(uw_hbm.at[idx_v.at[j]], rows_v, sem).wait()
      pltpu.sync_copy(rows_v, acc_s.at[seg_v.at[j]], add=True)
      pltpu.sync_copy(ones_v, cnt_s.at[seg_v.at[j]], add=True)
      return carry
    lax.fori_loop(0, NCHUNK, bag_chunk, 0)

    plsc.subcore_barrier()

    pltpu.sync_copy(cnt_s.at[pl.ds(s * SEG_PER_TILE, SEG_PER_TILE)], stage1d_v)
    pltpu.sync_copy(stage1d_v, pcnt_hbm.at[c, pl.ds(s * SEG_PER_TILE,
                                                    SEG_PER_TILE)])
    for k in range(SEG_PER_TILE // 256):
      off0 = s * SEG_PER_TILE + k * 256
      pltpu.sync_copy(acc_s.at[pl.ds(off0, 256)], stage2d_v)
      pltpu.sync_copy(stage2d_v, psum_hbm.at[c, pl.ds(off0, 256)])

  return body(bag2d, offsets, u_weight)


def _sc_v_kernel(v2d, vdst2d, v_weight):
  mesh = plsc.VectorSubcoreMesh(core_axis_name="c", subcore_axis_name="s")

  @functools.partial(
      pl.kernel,
      out_type=jax.ShapeDtypeStruct((VTOT * 4, D), jnp.float32),
      mesh=mesh,
      compiler_params=pltpu.CompilerParams(needs_layout_passes=False,
                                           use_tc_tiling_on_sc=False),
      scratch_types=(
          pltpu.VMEM((NVCHUNK, CHUNK), jnp.int32),
          pltpu.VMEM((NVCHUNK, CHUNK), jnp.int32),
          pltpu.VMEM((2, CHUNK, D), jnp.float32),
          pltpu.SemaphoreType.DMA,
          pltpu.SemaphoreType.DMA,
      ),
  )
  def body(v_hbm, vdst_hbm, vw_hbm, embv_hbm, vidx_v, vdst_v, vrows_v,
           gsem, ssem):
    c = lax.axis_index("c")
    s = lax.axis_index("s")
    wid = s * NC + c
    pltpu.sync_copy(v_hbm.at[pl.ds(wid * NVCHUNK, NVCHUNK)], vidx_v)
    pltpu.sync_copy(vdst_hbm.at[pl.ds(wid * NVCHUNK, NVCHUNK)], vdst_v)

    pltpu.async_copy(vw_hbm.at[vidx_v.at[0]], vrows_v.at[0], gsem)

    def v_chunk(j, carry):
      pltpu.make_async_copy(vw_hbm.at[vidx_v.at[j]], vrows_v.at[j % 2],
                            gsem).wait()

      @pl.when(j > 0)
      def _():
        pltpu.make_async_copy(vrows_v.at[(j - 1) % 2],
                              embv_hbm.at[vdst_v.at[j - 1]], ssem).wait()

      @pl.when(j < NVCHUNK - 1)
      def _():
        pltpu.async_copy(vw_hbm.at[vidx_v.at[j + 1]],
                         vrows_v.at[(j + 1) % 2], gsem)

      pltpu.async_copy(vrows_v.at[j % 2], embv_hbm.at[vdst_v.at[j]], ssem)
      return carry
    lax.fori_loop(0, NVCHUNK, v_chunk, 0)
    pltpu.make_async_copy(vrows_v.at[(NVCHUNK - 1) % 2],
                          embv_hbm.at[vdst_v.at[NVCHUNK - 1]], ssem).wait()

  return body(v2d, vdst2d, v_weight)


TBLK = 16384
TT = TBLK // 4
_SH = TT.bit_length() - 1


def _tr_body(in_ref, out_ref):
  x = in_ref[...]
  out_ref[...] = jnp.concatenate(
      [x[:, g * TT:(g + 1) * TT] for g in range(4)], axis=0).T


def _tc_transpose(table_t):
  V = table_t.shape[1]
  nblk = (V + TBLK - 1) // TBLK
  return pl.pallas_call(
      _tr_body,
      grid=(nblk,),
      in_specs=[pl.BlockSpec((D, TBLK), lambda i: (0, i))],
      out_specs=pl.BlockSpec((TT, 128), lambda i: (i, 0)),
      out_shape=jax.ShapeDtypeStruct((nblk * TT, 128), jnp.float32),
  )(table_t)


def _perm(r):
  return (((r >> (_SH + 2)) << (_SH + 2)) + ((r & (TT - 1)) << 2)
          + ((r >> _SH) & 3))


BLK = 1024


def _tc_body(ps_ref, pc_ref, ev_ref, out_ref):
  ps = ps_ref[...]
  pc = pc_ref[...]
  ev = ev_ref[...][:, :, :D]
  sums = ps[0] + ps[1]
  cnt = pc[0] + pc[1]
  emb_u = sums / jnp.maximum(cnt, 1.0)[:, None]
  out_ref[...] = jnp.sum(emb_u[None, :, :] * ev, axis=-1)


def _tc_dot(psum, pcnt, embv_g):
  nblk = B // BLK
  return pl.pallas_call(
      _tc_body,
      grid=(nblk,),
      in_specs=[
          pl.BlockSpec((NC, BLK, D), lambda i: (0, i, 0)),
          pl.BlockSpec((NC, BLK), lambda i: (0, i)),
          pl.BlockSpec((NSAMP, BLK, 128), lambda i: (0, i, 0)),
      ],
      out_specs=pl.BlockSpec((NSAMP, BLK), lambda i: (0, i)),
      out_shape=jax.ShapeDtypeStruct((NSAMP, B), jnp.float32),
  )(psum, pcnt, embv_g)


@jax.jit
def kernel(bag, offsets, v, u_weight, v_weight):
  bag2d = _perm(bag.astype(jnp.int32)).reshape(NW * NCHUNK, CHUNK)
  v2d = _perm(v.T.astype(jnp.int32)).reshape(NW * NVCHUNK, CHUNK)
  vdst2d = (jnp.arange(VTOT, dtype=jnp.int32) * 4).reshape(
      NW * NVCHUNK, CHUNK)
  uw = _tc_transpose(u_weight.T)
  uw = uw.reshape(uw.shape[0] * 4, D)
  psum, pcnt = _sc_u_kernel(bag2d, offsets.astype(jnp.int32), uw)
  vw = _tc_transpose(v_weight.T)
  vw = vw.reshape(vw.shape[0] * 4, D)
  embv4 = _sc_v_kernel(v2d, vdst2d, vw)
  s6 = _tc_dot(psum, pcnt, embv4.reshape(NSAMP, B, 128))
  return s6.T
